# baseline (speedup 1.0000x reference)
import functools

import jax
import jax.numpy as jnp
from jax import lax
from jax.experimental import pallas as pl
from jax.experimental.pallas import tpu as pltpu
from jax.experimental.pallas import tpu_sc as plsc

ROWS = 16384
COLS = 128
NCORES = 2
NSUBCORES = 16
NW = NCORES * NSUBCORES
RPW = ROWS // NW
LANES = 16
WORDS = RPW * 4
GROUPS = WORDS // LANES
DMA_CHUNK = 128
NCHUNKS = WORDS // DMA_CHUNK

_mesh = plsc.VectorSubcoreMesh(core_axis_name="c", subcore_axis_name="s")


@functools.partial(
    pl.kernel,
    out_type=jax.ShapeDtypeStruct((ROWS * 4,), jnp.float32),
    mesh=_mesh,
    scratch_types=[
        pltpu.VMEM((WORDS,), jnp.int32),
        pltpu.VMEM((WORDS,), jnp.float32),
        pltpu.VMEM((WORDS,), jnp.float32),
        pltpu.SemaphoreType.DMA,
    ],
)
def _deepjets_sc(x_hbm, out_hbm, idx_v, gat_v, out_v, sem):
    wid = lax.axis_index("s") * NCORES + lax.axis_index("c")
    base_row = wid * RPW
    lane = lax.iota(jnp.int32, LANES)
    rowoff = lane >> 2
    colo = lane & 3
    lane_base = lane & ~3

    def fill_body(j, carry):
        idx_v[pl.ds(j * LANES, LANES)] = (
            (base_row + j * 4 + rowoff) * COLS + 4 + colo
        )
        return carry

    lax.fori_loop(0, GROUPS, fill_body, 0)

    copies = [
        pltpu.async_copy(
            x_hbm.at[idx_v.at[pl.ds(k * DMA_CHUNK, DMA_CHUNK)]],
            gat_v.at[pl.ds(k * DMA_CHUNK, DMA_CHUNK)],
            sem,
        )
        for k in range(NCHUNKS)
    ]
    for cp in copies:
        cp.wait()

    is0 = colo == 0
    is1 = colo == 1
    is2 = colo == 2

    def comp_body(j, carry):
        v = gat_v[pl.ds(j * LANES, LANES)]
        b = v.at[lane_base].get(mode="promise_in_bounds")
        cvb = v.at[lane_base + 1].get(mode="promise_in_bounds")
        cvl = v.at[lane_base + 2].get(mode="promise_in_bounds")
        qg = v.at[lane_base + 3].get(mode="promise_in_bounds")
        c = b / (1.0 / cvb - 1.0)
        d = c / cvl - c
        out = jnp.where(is0, b, jnp.where(is1, c, jnp.where(is2, (1.0 - qg) * d, qg * d)))
        out_v[pl.ds(j * LANES, LANES)] = out
        return carry

    lax.fori_loop(0, GROUPS, comp_body, 0)

    pltpu.sync_copy(out_v, out_hbm.at[pl.ds(wid * WORDS, WORDS)])


def kernel(x):
    x_flat = x.reshape(ROWS * COLS)
    return _deepjets_sc(x_flat).reshape(ROWS, 4)
